# baseline (speedup 1.0000x reference)
import functools

import jax
import jax.numpy as jnp
from jax import lax
from jax.experimental import pallas as pl
from jax.experimental.pallas import tpu as pltpu
from jax.experimental.pallas import tpu_sc as plsc

VOCAB = 100000
HIDDEN = 512
EMBED = 128

M_BLK = 10000

NC = 2
NS = 16
NW = NC * NS
NBUF = 4


def _matmul_body(base_ref, w_ref, out_ref):
    out_ref[...] = jnp.dot(base_ref[...], w_ref[...],
                           preferred_element_type=jnp.float32)


def _compose_table(base, W):
    grid = VOCAB // M_BLK
    return pl.pallas_call(
        _matmul_body,
        grid=(grid,),
        in_specs=[
            pl.BlockSpec((M_BLK, HIDDEN), lambda i: (i, 0)),
            pl.BlockSpec((HIDDEN, EMBED), lambda i: (0, 0)),
        ],
        out_specs=pl.BlockSpec((M_BLK, EMBED), lambda i: (i, 0)),
        out_shape=jax.ShapeDtypeStruct((VOCAB, EMBED), jnp.float32),
    )(base, W)


def _make_gather(bsz, seq):
    b_per_w = bsz // NW
    n_chunks = seq
    mesh = plsc.VectorSubcoreMesh(core_axis_name="c", subcore_axis_name="s")

    @functools.partial(
        pl.kernel,
        mesh=mesh,
        out_type=jax.ShapeDtypeStruct((seq, bsz, EMBED), jnp.float32),
        scratch_types=[
            pltpu.VMEM((n_chunks, b_per_w), jnp.int32),
        ] + [pltpu.VMEM((b_per_w, EMBED), jnp.float32)
             for _ in range(NBUF)]
        + [pltpu.SemaphoreType.DMA for _ in range(2 * NBUF)],
    )
    def gather_k(table_hbm, idx_hbm, out_hbm, idx_v, *bufs_sems):
        bufs = bufs_sems[:NBUF]
        gsems = bufs_sems[NBUF:2 * NBUF]
        osems = bufs_sems[2 * NBUF:]

        wid = lax.axis_index("s") * NC + lax.axis_index("c")
        b0 = wid * b_per_w
        pltpu.sync_copy(idx_hbm.at[wid], idx_v)

        def fire(c, b):
            pltpu.async_copy(table_hbm.at[idx_v.at[c]], bufs[b], gsems[b])

        def drain(c, b):
            pltpu.make_async_copy(table_hbm.at[idx_v.at[c]], bufs[b],
                                  gsems[b]).wait()

        def fire_write(c, b):
            pltpu.async_copy(bufs[b], out_hbm.at[c, pl.ds(b0, b_per_w)],
                             osems[b])

        def wait_write(c, b):
            pltpu.make_async_copy(bufs[b], out_hbm.at[c, pl.ds(b0, b_per_w)],
                                  osems[b]).wait()

        def on_slot(slot, fn):
            for b in range(NBUF):
                @pl.when(slot == b)
                def _():
                    fn(b)

        for c in range(NBUF - 1):
            fire(c, c)

        def body(g, _):
            slot = lax.rem(g, NBUF)
            on_slot(slot, lambda b: drain(g, b))
            on_slot(slot, lambda b: fire_write(g, b))

            @pl.when(g + NBUF - 1 < n_chunks)
            def _():
                nslot = lax.rem(g + NBUF - 1, NBUF)

                @pl.when(g >= 1)
                def _():
                    on_slot(nslot, lambda b: wait_write(g - 1, b))
                on_slot(nslot, lambda b: fire(g + NBUF - 1, b))
            return 0

        lax.fori_loop(0, n_chunks, body, 0)

        for c in range(n_chunks - NBUF, n_chunks):
            wait_write(c, c % NBUF)

    return gather_k


def kernel(data, base, W):
    d = jnp.squeeze(data, axis=2)
    bsz, seq = d.shape
    idx = (d.astype(jnp.int32)
           .T.reshape(seq, NW, bsz // NW)
           .transpose(1, 0, 2))

    var = _compose_table(base, W)
    out = _make_gather(bsz, seq)(var, idx)
    return jnp.transpose(out, (1, 0, 2))


# speedup vs baseline: 1.0077x; 1.0077x over previous
import functools

import jax
import jax.numpy as jnp
from jax import lax
from jax.experimental import pallas as pl
from jax.experimental.pallas import tpu as pltpu
from jax.experimental.pallas import tpu_sc as plsc

VOCAB = 100000
HIDDEN = 512
EMBED = 128

M_BLK = 5000

NC = 2
NS = 16
NW = NC * NS
NBUF = 4


def _matmul_body(base_ref, w_ref, out_ref):
    out_ref[...] = jnp.dot(base_ref[...], w_ref[...],
                           preferred_element_type=jnp.float32)


def _compose_table(base, W):
    grid = VOCAB // M_BLK
    return pl.pallas_call(
        _matmul_body,
        grid=(grid,),
        in_specs=[
            pl.BlockSpec((M_BLK, HIDDEN), lambda i: (i, 0)),
            pl.BlockSpec((HIDDEN, EMBED), lambda i: (0, 0)),
        ],
        out_specs=pl.BlockSpec((M_BLK, EMBED), lambda i: (i, 0)),
        out_shape=jax.ShapeDtypeStruct((VOCAB, EMBED), jnp.float32),
    )(base, W)


def _make_gather(bsz, seq):
    b_per_w = bsz // NW
    n_chunks = seq
    mesh = plsc.VectorSubcoreMesh(core_axis_name="c", subcore_axis_name="s")

    @functools.partial(
        pl.kernel,
        mesh=mesh,
        out_type=jax.ShapeDtypeStruct((seq, bsz, EMBED), jnp.float32),
        scratch_types=[
            pltpu.VMEM((n_chunks, b_per_w), jnp.int32),
        ] + [pltpu.VMEM((b_per_w, EMBED), jnp.float32)
             for _ in range(NBUF)]
        + [pltpu.SemaphoreType.DMA for _ in range(2 * NBUF)],
    )
    def gather_k(table_hbm, idx_hbm, out_hbm, idx_v, *bufs_sems):
        bufs = bufs_sems[:NBUF]
        gsems = bufs_sems[NBUF:2 * NBUF]
        osems = bufs_sems[2 * NBUF:]

        wid = lax.axis_index("s") * NC + lax.axis_index("c")
        b0 = wid * b_per_w
        pltpu.sync_copy(idx_hbm.at[wid], idx_v)

        def fire(c, b):
            pltpu.async_copy(table_hbm.at[idx_v.at[c]], bufs[b], gsems[b])

        def drain(c, b):
            pltpu.make_async_copy(table_hbm.at[idx_v.at[c]], bufs[b],
                                  gsems[b]).wait()

        def fire_write(c, b):
            pltpu.async_copy(bufs[b], out_hbm.at[c, pl.ds(b0, b_per_w)],
                             osems[b])

        def wait_write(c, b):
            pltpu.make_async_copy(bufs[b], out_hbm.at[c, pl.ds(b0, b_per_w)],
                                  osems[b]).wait()

        def on_slot(slot, fn):
            for b in range(NBUF):
                @pl.when(slot == b)
                def _():
                    fn(b)

        for c in range(NBUF - 1):
            fire(c, c)

        def body(g, _):
            slot = lax.rem(g, NBUF)
            on_slot(slot, lambda b: drain(g, b))
            on_slot(slot, lambda b: fire_write(g, b))

            @pl.when(g + NBUF - 1 < n_chunks)
            def _():
                nslot = lax.rem(g + NBUF - 1, NBUF)

                @pl.when(g >= 1)
                def _():
                    on_slot(nslot, lambda b: wait_write(g - 1, b))
                on_slot(nslot, lambda b: fire(g + NBUF - 1, b))
            return 0

        lax.fori_loop(0, n_chunks, body, 0)

        for c in range(n_chunks - NBUF, n_chunks):
            wait_write(c, c % NBUF)

    return gather_k


def kernel(data, base, W):
    d = jnp.squeeze(data, axis=2)
    bsz, seq = d.shape
    idx = (d.astype(jnp.int32)
           .T.reshape(seq, NW, bsz // NW)
           .transpose(1, 0, 2))

    var = _compose_table(base, W)
    out = _make_gather(bsz, seq)(var, idx)
    return jnp.transpose(out, (1, 0, 2))
